# baseline (speedup 1.0000x reference)
import functools
import jax
import jax.numpy as jnp
import numpy as np
from jax import lax
from jax.experimental import pallas as pl
from jax.experimental.pallas import tpu as pltpu
from jax.experimental.pallas import tpu_sc as plsc

N_NODES = 100000
N_EDGES = 1600000
N_GRAPHS = 64
KTOP = 100
NS = 16
NCORES = 2
CHUNK = 1024
NROW = CHUNK // 128
EPT = N_EDGES // NS
NCHUNK = -(-EPT // CHUNK)
PEPT = NCHUNK * CHUNK
HALF = 16
TROWS = 2 * N_NODES + 8


def _agg_body(tab_hbm, src_hbm, dst_hbm, zeros_hbm, out_hbm,
              src_v, dst_v, rows_v, acc, sem):
    c = lax.axis_index("c")
    s = lax.axis_index("s")

    @pl.when(s == 0)
    def _zero():
        pltpu.sync_copy(zeros_hbm, acc)

    plsc.subcore_barrier()

    def body(j, carry):
        srow = ((c * NS + s) * NCHUNK + j) * NROW
        drow = (s * NCHUNK + j) * NROW
        pltpu.sync_copy(src_hbm.at[pl.ds(srow, NROW)], src_v)
        pltpu.sync_copy(dst_hbm.at[pl.ds(drow, NROW)], dst_v)
        for k in range(NROW):
            pltpu.async_copy(tab_hbm.at[src_v.at[k]],
                             rows_v.at[pl.ds(k * 128, 128)], sem).wait()
            pltpu.sync_copy(rows_v.at[pl.ds(k * 128, 128)],
                            acc.at[dst_v.at[k]], add=True)
        return carry

    lax.fori_loop(0, NCHUNK, body, 0)
    plsc.subcore_barrier()

    @pl.when(s == 0)
    def _drain():
        pltpu.sync_copy(acc, out_hbm.at[pl.ds(c * N_NODES, N_NODES)])


@functools.cache
def _make_agg_call():
    return functools.partial(
        pl.kernel,
        mesh=plsc.VectorSubcoreMesh(core_axis_name="c", subcore_axis_name="s"),
        compiler_params=pltpu.CompilerParams(use_tc_tiling_on_sc=False),
        out_type=jax.ShapeDtypeStruct((2 * N_NODES, HALF), jnp.float32),
        scratch_types=[
            pltpu.VMEM((NROW, 128), jnp.int32),
            pltpu.VMEM((NROW, 128), jnp.int32),
            pltpu.VMEM((CHUNK, HALF), jnp.float32),
            pltpu.VMEM_SHARED((N_NODES, HALF), jnp.float32),
            pltpu.SemaphoreType.DMA,
        ],
    )(_agg_body)


def _sc_aggregate(g, src2, dstp, zeros):
    tab = jnp.concatenate(
        [g[:, :HALF], g[:, HALF:], jnp.zeros((8, HALF), jnp.float32)], axis=0)
    out = _make_agg_call()(tab, src2, dstp, zeros)
    return jnp.concatenate([out[:N_NODES], out[N_NODES:]], axis=1)


NB = 2000
NGRID = N_NODES // NB


def _pre_body(x_ref, dinv_ref, W_ref, g_ref):
    g_ref[...] = dinv_ref[...] * jnp.dot(
        x_ref[...], W_ref[...], preferred_element_type=jnp.float32)


def _tc_pre(x, dinv2, W):
    din = x.shape[1]
    return pl.pallas_call(
        _pre_body,
        grid=(NGRID,),
        in_specs=[
            pl.BlockSpec((NB, din), lambda i: (i, 0)),
            pl.BlockSpec((NB, 1), lambda i: (i, 0)),
            pl.BlockSpec((din, 32), lambda i: (0, 0)),
        ],
        out_specs=pl.BlockSpec((NB, 32), lambda i: (i, 0)),
        out_shape=jax.ShapeDtypeStruct((N_NODES, 32), jnp.float32),
    )(x, dinv2, W)


def _post_body(s_ref, g_ref, dinv_ref, b_ref, W_ref, x_ref, gn_ref):
    xn = jnp.tanh(dinv_ref[...] * (s_ref[...] + g_ref[...]) + b_ref[...])
    x_ref[...] = xn
    gn_ref[...] = dinv_ref[...] * jnp.dot(
        xn, W_ref[...], preferred_element_type=jnp.float32)


def _tc_post(s, g, dinv2, b, Wn):
    return pl.pallas_call(
        _post_body,
        grid=(NGRID,),
        in_specs=[
            pl.BlockSpec((NB, 32), lambda i: (i, 0)),
            pl.BlockSpec((NB, 32), lambda i: (i, 0)),
            pl.BlockSpec((NB, 1), lambda i: (i, 0)),
            pl.BlockSpec((1, 32), lambda i: (0, 0)),
            pl.BlockSpec((32, 32), lambda i: (0, 0)),
        ],
        out_specs=[pl.BlockSpec((NB, 32), lambda i: (i, 0)),
                   pl.BlockSpec((NB, 32), lambda i: (i, 0))],
        out_shape=[jax.ShapeDtypeStruct((N_NODES, 32), jnp.float32),
                   jax.ShapeDtypeStruct((N_NODES, 32), jnp.float32)],
    )(s, g, dinv2, b.reshape(1, 32), Wn)


def _final_body(x1_ref, x2_ref, s_ref, g_ref, dinv_ref, W3_ref, b3_ref,
                W4_ref, b4_ref, h_ref):
    a2 = dinv_ref[...] * (s_ref[...] + g_ref[...])
    x3 = jnp.tanh(jnp.dot(a2, W3_ref[...],
                          preferred_element_type=jnp.float32) + b3_ref[...])
    x4 = jnp.tanh(jnp.dot(a2, W4_ref[...],
                          preferred_element_type=jnp.float32) + b4_ref[...])
    h_ref[...] = jnp.concatenate([x1_ref[...], x2_ref[...], x3, x4], axis=1)


def _tc_final(x1, x2, s3, g3, dinv2, W3, b3, W4, b4):
    return pl.pallas_call(
        _final_body,
        grid=(NGRID,),
        in_specs=[
            pl.BlockSpec((NB, 32), lambda i: (i, 0)),
            pl.BlockSpec((NB, 32), lambda i: (i, 0)),
            pl.BlockSpec((NB, 32), lambda i: (i, 0)),
            pl.BlockSpec((NB, 32), lambda i: (i, 0)),
            pl.BlockSpec((NB, 1), lambda i: (i, 0)),
            pl.BlockSpec((32, 32), lambda i: (0, 0)),
            pl.BlockSpec((1, 32), lambda i: (0, 0)),
            pl.BlockSpec((32, 1), lambda i: (0, 0)),
            pl.BlockSpec((1, 1), lambda i: (0, 0)),
        ],
        out_specs=pl.BlockSpec((NB, 97), lambda i: (i, 0)),
        out_shape=jax.ShapeDtypeStruct((N_NODES, 97), jnp.float32),
    )(x1, x2, s3, g3, dinv2, W3, b3.reshape(1, 32), W4, b4.reshape(1, 1))


def _headA_body(p_ref, w_ref, b_ref, m_ref):
    r1 = jax.nn.relu(jnp.dot(p_ref[...], w_ref[...],
                             preferred_element_type=jnp.float32) + b_ref[...])
    m_ref[...] = jnp.max(r1.reshape(r1.shape[0] // 2, 2, 128), axis=1)


def _tc_headA(p2, cw1r, cb1):
    return pl.pallas_call(
        _headA_body,
        out_shape=jax.ShapeDtypeStruct((N_GRAPHS * KTOP // 2, 128),
                                       jnp.float32),
    )(p2, cw1r, cb1.reshape(1, 128))


def _headB_body(w_ref, wt_ref, b_ref, o_ref):
    o_ref[...] = jax.nn.relu(
        jnp.dot(w_ref[...], wt_ref[...],
                preferred_element_type=jnp.float32) + b_ref[...])


def _tc_headB(wins, w2flat, cb2):
    return pl.pallas_call(
        _headB_body,
        out_shape=jax.ShapeDtypeStruct((wins.shape[0], 64), jnp.float32),
    )(wins, w2flat, cb2.reshape(1, 64))


def _headC_body(f_ref, w1_ref, b1_ref, w3_ref, b3_ref, o_ref):
    h1 = jax.nn.relu(jnp.dot(f_ref[...], w1_ref[...],
                             preferred_element_type=jnp.float32) + b1_ref[...])
    z = jnp.dot(h1, w3_ref[...],
                preferred_element_type=jnp.float32) + b3_ref[...]
    zmax = jnp.max(z, axis=1, keepdims=True)
    zs = z - zmax
    o_ref[...] = zs - jnp.log(jnp.sum(jnp.exp(zs), axis=1, keepdims=True))


def _tc_headC(flat, fc1_W, fc1_b, fc3_W, fc3_b):
    return pl.pallas_call(
        _headC_body,
        out_shape=jax.ShapeDtypeStruct((N_GRAPHS, 2), jnp.float32),
    )(flat, fc1_W, fc1_b.reshape(1, 128), fc3_W, fc3_b.reshape(1, 2))


def _global_sort_pool(h, batch, k, num_graphs):
    n, d = h.shape
    counts = jnp.bincount(batch, length=num_graphs)
    starts = jnp.concatenate([jnp.zeros((1,), counts.dtype),
                              jnp.cumsum(counts)[:-1]])
    perm = jnp.lexsort((-h[:, -1], batch))
    hs = h[perm]
    idx = starts[:, None] + jnp.arange(k)[None, :]
    valid = jnp.arange(k)[None, :] < counts[:, None]
    gathered = hs[jnp.clip(idx, 0, n - 1)]
    pooled = jnp.where(valid[:, :, None], gathered, 0.0)
    return pooled.reshape(num_graphs, k * d)


def kernel(x, edge_index, batch, W1, b1, W2, b2, W3, b3, W4, b4,
           cw1, cb1, cw2, cb2, fc1_W, fc1_b, fc3_W, fc3_b):
    src = edge_index[0].astype(jnp.int32)
    dst = edge_index[1].astype(jnp.int32)

    npad = PEPT - EPT
    srcr = src.reshape(NS, EPT)
    padi = jnp.full((NS, npad), 2 * N_NODES, jnp.int32)
    s0 = jnp.concatenate([srcr, padi], axis=1)
    s1 = jnp.concatenate([srcr + N_NODES, padi], axis=1)
    src2 = jnp.stack([s0, s1]).reshape(-1, 128)
    dstp = jnp.concatenate(
        [dst.reshape(NS, EPT), jnp.zeros((NS, npad), jnp.int32)],
        axis=1).reshape(-1, 128)
    zeros = jnp.zeros((N_NODES, HALF), jnp.float32)

    ones_tab = jnp.ones((N_NODES, 2 * HALF), jnp.float32)
    indeg = _sc_aggregate(ones_tab, src2, dstp, zeros)[:, 0]
    dinv2 = lax.rsqrt(indeg + 1.0).reshape(N_NODES, 1)

    g1 = _tc_pre(x, dinv2, W1)
    s1 = _sc_aggregate(g1, src2, dstp, zeros)
    x1, g2 = _tc_post(s1, g1, dinv2, b1, W2)
    s2 = _sc_aggregate(g2, src2, dstp, zeros)
    x2, g3 = _tc_post(s2, g2, dinv2, b2, jnp.eye(32, dtype=jnp.float32))
    s3 = _sc_aggregate(g3, src2, dstp, zeros)
    h = _tc_final(x1, x2, s3, g3, dinv2, W3, b3, W4, b4)
    p = _global_sort_pool(h, batch, KTOP, N_GRAPHS)

    p2 = p.reshape(N_GRAPHS * KTOP, 97)
    m = _tc_headA(p2, cw1r=cw1.reshape(128, 97).T, cb1=cb1)
    m3 = m.reshape(N_GRAPHS, KTOP // 2, 128)
    wins = jnp.concatenate([m3[:, k:k + 46, :] for k in range(5)], axis=2)
    w2flat = cw2.transpose(2, 1, 0).reshape(640, 64)
    r2 = _tc_headB(wins.reshape(N_GRAPHS * 46, 640), w2flat, cb2)
    flat = r2.reshape(N_GRAPHS, 46, 64).transpose(0, 2, 1).reshape(N_GRAPHS, -1)
    return _tc_headC(flat, fc1_W, fc1_b, fc3_W, fc3_b)
